# baseline (speedup 1.0000x reference)
import functools

import jax
import jax.numpy as jnp
from jax import lax
from jax.experimental import pallas as pl
from jax.experimental.pallas import tpu as pltpu
from jax.experimental.pallas import tpu_sc as plsc

KCONST = 10.0
NSEG = 1024
TOTAL = 32768
NC = 2
NS = 16
L = 16
NW = NC * NS
CHUNK = TOTAL // NW
NVEC = CHUNK // L
SEG_PER_TILE = NSEG // NS


def _sc_segsum_body(xf_hbm, idx_hbm, out_hbm,
                    chunk_v, idx_v, a1buf, acc0, acc1, sh0, sh1, red, outv,
                    sem):
    cid = lax.axis_index("c")
    sid = lax.axis_index("s")
    wid = cid * NS + sid
    base = wid * CHUNK

    h1 = pltpu.async_copy(xf_hbm.at[pl.ds(3 * base, 3 * CHUNK)], chunk_v, sem)
    h2 = pltpu.async_copy(idx_hbm.at[pl.ds(base, CHUNK)],
                          idx_v.at[pl.ds(L, CHUNK)], sem)

    neg1 = jnp.full((L,), -1, jnp.int32)
    idx_v[pl.ds(0, L)] = neg1
    idx_v[pl.ds(L + CHUNK, L)] = neg1
    zf = jnp.zeros((L,), jnp.float32)
    for i in range(NVEC):
        acc0[pl.ds(i * L, L)] = zf
        acc1[pl.ds(i * L, L)] = zf
    h1.wait()
    h2.wait()

    g0 = lax.iota(jnp.int32, L) * 3

    tot0 = []
    tot1 = []
    for i in range(NVEC):
        o = 48 * i
        b0 = plsc.load_gather(chunk_v, [g0 + o])
        b1 = plsc.load_gather(chunk_v, [g0 + (o + 1)])
        b2 = plsc.load_gather(chunk_v, [g0 + (o + 2)])
        a1 = b1 + b2
        a1buf[pl.ds(i * L, L)] = a1
        tot0.append(jnp.sum(b0))
        tot1.append(jnp.sum(a1))

    c0 = jnp.float32(0.0)
    c1 = jnp.float32(0.0)
    car0 = []
    car1 = []
    for i in range(NVEC):
        car0.append(c0)
        car1.append(c1)
        c0 = c0 + tot0[i]
        c1 = c1 + tot1[i]

    for i in range(NVEC):
        off = i * L
        ids = idx_v[pl.ds(L + off, L)]
        nxt = idx_v[pl.ds(L + off + 1, L)]
        a0 = plsc.load_gather(chunk_v, [g0 + 48 * i])
        a1 = a1buf[pl.ds(off, L)]
        p0 = plsc.cumsum(a0) + car0[i]
        p1 = plsc.cumsum(a1) + car1[i]
        endm = ids != nxt
        stm = jnp.logical_and(endm, nxt >= 0)
        plsc.addupdate_scatter(acc0, [ids], p0, mask=endm)
        plsc.addupdate_scatter(acc0, [nxt], -p0, mask=stm)
        plsc.addupdate_scatter(acc1, [ids], p1, mask=endm)
        plsc.addupdate_scatter(acc1, [nxt], -p1, mask=stm)

    pltpu.sync_copy(acc0, sh0.at[pl.ds(sid * NSEG, NSEG)])
    pltpu.sync_copy(acc1, sh1.at[pl.ds(sid * NSEG, NSEG)])
    plsc.subcore_barrier()

    seg0 = sid * SEG_PER_TILE
    hs = []
    for k in range(NS):
        hs.append(pltpu.async_copy(
            sh0.at[pl.ds(k * NSEG + seg0, SEG_PER_TILE)],
            red.at[pl.ds(k * SEG_PER_TILE, SEG_PER_TILE)], sem))
        hs.append(pltpu.async_copy(
            sh1.at[pl.ds(k * NSEG + seg0, SEG_PER_TILE)],
            red.at[pl.ds(NSEG + k * SEG_PER_TILE, SEG_PER_TILE)], sem))
    for h in hs:
        h.wait()
    for s in range(2):
        sb = s * NSEG
        for j in range(SEG_PER_TILE // L):
            t = red[pl.ds(sb + j * L, L)]
            for k in range(1, NS):
                t = t + red[pl.ds(sb + k * SEG_PER_TILE + j * L, L)]
            outv[pl.ds(s * SEG_PER_TILE + j * L, L)] = t
    pltpu.sync_copy(outv.at[pl.ds(0, SEG_PER_TILE)],
                    out_hbm.at[pl.ds(cid * 2 * NSEG + seg0, SEG_PER_TILE)])
    pltpu.sync_copy(outv.at[pl.ds(SEG_PER_TILE, SEG_PER_TILE)],
                    out_hbm.at[pl.ds((cid * 2 + 1) * NSEG + seg0,
                                     SEG_PER_TILE)])


_sc_segsum = functools.partial(
    pl.kernel,
    out_type=jax.ShapeDtypeStruct((4 * NSEG,), jnp.float32),
    mesh=plsc.VectorSubcoreMesh(core_axis_name="c", subcore_axis_name="s"),
    compiler_params=pltpu.CompilerParams(needs_layout_passes=False),
    scratch_types=[
        pltpu.VMEM((3 * CHUNK,), jnp.float32),
        pltpu.VMEM((2 * L + CHUNK,), jnp.int32),
        pltpu.VMEM((CHUNK,), jnp.float32),
        pltpu.VMEM((NSEG,), jnp.float32),
        pltpu.VMEM((NSEG,), jnp.float32),
        pltpu.VMEM_SHARED((NS * NSEG,), jnp.float32),
        pltpu.VMEM_SHARED((NS * NSEG,), jnp.float32),
        pltpu.VMEM((2 * NS * SEG_PER_TILE,), jnp.float32),
        pltpu.VMEM((2 * SEG_PER_TILE,), jnp.float32),
        pltpu.SemaphoreType.DMA,
    ],
)(_sc_segsum_body)


def _tc_tail_body(x_ref, o_ref):
    x = x_ref[...].reshape(4, NSEG)
    s0 = x[0, :] + x[2, :]
    s1 = x[1, :] + x[3, :]
    p1 = 1.0 / (1.0 + jnp.exp(-KCONST * (1.0 - s1)))
    p0 = 1.0 / (1.0 + jnp.exp(-KCONST * (5.0 - s0)))
    o_ref[0, :] = jnp.log(p1 + 1e-10)
    o_ref[1, :] = jnp.log(p0 + 1e-10)


_tc_tail = pl.pallas_call(
    _tc_tail_body,
    out_shape=jax.ShapeDtypeStruct((2, NSEG), jnp.float32),
)


def kernel(sub_logits, original_indices):
    xf = sub_logits.reshape(-1)
    partials = _sc_segsum(xf, original_indices)
    out2 = _tc_tail(partials)
    return out2.T


# speedup vs baseline: 1.5864x; 1.5864x over previous
import functools

import jax
import jax.numpy as jnp
from jax import lax
from jax.experimental import pallas as pl
from jax.experimental.pallas import tpu as pltpu
from jax.experimental.pallas import tpu_sc as plsc

KCONST = 10.0
NSEG = 1024
TOTAL = 32768
NC = 2
NS = 16
L = 16
NW = NC * NS
CHUNK = TOTAL // NW
NVEC = CHUNK // L
SEG_PER_TILE = NSEG // NS


def _sc_segsum_body(xf_hbm, idx_hbm, out_hbm,
                    chunk_v, idx_v, a1buf, acc0, acc1, sh0, sh1, red, outv,
                    sem):
    cid = lax.axis_index("c")
    sid = lax.axis_index("s")
    wid = cid * NS + sid
    base = wid * CHUNK

    h0 = pltpu.async_copy(xf_hbm.at[pl.ds(base, CHUNK)],
                          chunk_v.at[pl.ds(0, CHUNK)], sem)
    h1 = pltpu.async_copy(xf_hbm.at[pl.ds(TOTAL + base, CHUNK)],
                          chunk_v.at[pl.ds(CHUNK, CHUNK)], sem)
    h2 = pltpu.async_copy(xf_hbm.at[pl.ds(2 * TOTAL + base, CHUNK)],
                          chunk_v.at[pl.ds(2 * CHUNK, CHUNK)], sem)
    h3 = pltpu.async_copy(idx_hbm.at[pl.ds(base, CHUNK)],
                          idx_v.at[pl.ds(L, CHUNK)], sem)

    neg1 = jnp.full((L,), -1, jnp.int32)
    idx_v[pl.ds(0, L)] = neg1
    idx_v[pl.ds(L + CHUNK, L)] = neg1
    zf = jnp.zeros((L,), jnp.float32)
    for i in range(NVEC):
        acc0[pl.ds(i * L, L)] = zf
        acc1[pl.ds(i * L, L)] = zf
    h0.wait()
    h1.wait()
    h2.wait()
    h3.wait()

    tot0 = []
    tot1 = []
    for i in range(NVEC):
        o = i * L
        b0 = chunk_v[pl.ds(o, L)]
        a1 = chunk_v[pl.ds(CHUNK + o, L)] + chunk_v[pl.ds(2 * CHUNK + o, L)]
        a1buf[pl.ds(o, L)] = a1
        tot0.append(jnp.sum(b0))
        tot1.append(jnp.sum(a1))

    c0 = jnp.float32(0.0)
    c1 = jnp.float32(0.0)
    car0 = []
    car1 = []
    for i in range(NVEC):
        car0.append(c0)
        car1.append(c1)
        c0 = c0 + tot0[i]
        c1 = c1 + tot1[i]

    for i in range(NVEC):
        off = i * L
        ids = idx_v[pl.ds(L + off, L)]
        nxt = idx_v[pl.ds(L + off + 1, L)]
        a0 = chunk_v[pl.ds(off, L)]
        a1 = a1buf[pl.ds(off, L)]
        p0 = plsc.cumsum(a0) + car0[i]
        p1 = plsc.cumsum(a1) + car1[i]
        endm = ids != nxt
        stm = jnp.logical_and(endm, nxt >= 0)
        plsc.addupdate_scatter(acc0, [ids], p0, mask=endm)
        plsc.addupdate_scatter(acc0, [nxt], -p0, mask=stm)
        plsc.addupdate_scatter(acc1, [ids], p1, mask=endm)
        plsc.addupdate_scatter(acc1, [nxt], -p1, mask=stm)

    pltpu.sync_copy(acc0, sh0.at[pl.ds(sid * NSEG, NSEG)])
    pltpu.sync_copy(acc1, sh1.at[pl.ds(sid * NSEG, NSEG)])
    plsc.subcore_barrier()

    seg0 = sid * SEG_PER_TILE
    hs = []
    for k in range(NS):
        hs.append(pltpu.async_copy(
            sh0.at[pl.ds(k * NSEG + seg0, SEG_PER_TILE)],
            red.at[pl.ds(k * SEG_PER_TILE, SEG_PER_TILE)], sem))
        hs.append(pltpu.async_copy(
            sh1.at[pl.ds(k * NSEG + seg0, SEG_PER_TILE)],
            red.at[pl.ds(NSEG + k * SEG_PER_TILE, SEG_PER_TILE)], sem))
    for h in hs:
        h.wait()
    for s in range(2):
        sb = s * NSEG
        for j in range(SEG_PER_TILE // L):
            t = red[pl.ds(sb + j * L, L)]
            for k in range(1, NS):
                t = t + red[pl.ds(sb + k * SEG_PER_TILE + j * L, L)]
            outv[pl.ds(s * SEG_PER_TILE + j * L, L)] = t
    pltpu.sync_copy(outv.at[pl.ds(0, SEG_PER_TILE)],
                    out_hbm.at[pl.ds(cid * 2 * NSEG + seg0, SEG_PER_TILE)])
    pltpu.sync_copy(outv.at[pl.ds(SEG_PER_TILE, SEG_PER_TILE)],
                    out_hbm.at[pl.ds((cid * 2 + 1) * NSEG + seg0,
                                     SEG_PER_TILE)])


_sc_segsum = functools.partial(
    pl.kernel,
    out_type=jax.ShapeDtypeStruct((4 * NSEG,), jnp.float32),
    mesh=plsc.VectorSubcoreMesh(core_axis_name="c", subcore_axis_name="s"),
    compiler_params=pltpu.CompilerParams(needs_layout_passes=False),
    scratch_types=[
        pltpu.VMEM((3 * CHUNK,), jnp.float32),
        pltpu.VMEM((2 * L + CHUNK,), jnp.int32),
        pltpu.VMEM((CHUNK,), jnp.float32),
        pltpu.VMEM((NSEG,), jnp.float32),
        pltpu.VMEM((NSEG,), jnp.float32),
        pltpu.VMEM_SHARED((NS * NSEG,), jnp.float32),
        pltpu.VMEM_SHARED((NS * NSEG,), jnp.float32),
        pltpu.VMEM((2 * NS * SEG_PER_TILE,), jnp.float32),
        pltpu.VMEM((2 * SEG_PER_TILE,), jnp.float32),
        pltpu.SemaphoreType.DMA,
    ],
)(_sc_segsum_body)


def _tc_tail_body(x_ref, o_ref):
    x = x_ref[...].reshape(4, NSEG)
    s0 = x[0, :] + x[2, :]
    s1 = x[1, :] + x[3, :]
    p1 = 1.0 / (1.0 + jnp.exp(-KCONST * (1.0 - s1)))
    p0 = 1.0 / (1.0 + jnp.exp(-KCONST * (5.0 - s0)))
    l1 = jnp.log(p1 + 1e-10)
    l0 = jnp.log(p0 + 1e-10)
    o_ref[...] = jnp.concatenate([l1[:, None], l0[:, None]], axis=1)


_tc_tail = pl.pallas_call(
    _tc_tail_body,
    out_shape=jax.ShapeDtypeStruct((NSEG, 2), jnp.float32),
)


def kernel(sub_logits, original_indices):
    vt = sub_logits.T.reshape(-1)
    partials = _sc_segsum(vt, original_indices)
    return _tc_tail(partials)
